# baseline (speedup 1.0000x reference)
import functools
import math

import jax
import jax.numpy as jnp
from jax import lax
from jax.experimental import pallas as pl
from jax.experimental.pallas import tpu as pltpu
from jax.experimental.pallas import tpu_sc as plsc

VOCAB = 100000
D = 128
B = 4096
S = 200
SEQ = S + 1
SCALE = 1.0 / math.sqrt(D)

_info = plsc.get_sparse_core_info()
NC, NS, L = _info.num_cores, _info.num_subcores, _info.num_lanes
NW = NC * NS
BPW = B // NW
CK = 128
NCHUNK = B // CK
UNITS = S * NCHUNK // NW
NSLOT = 5


def _make_sc_gather():
    mesh = plsc.VectorSubcoreMesh(core_axis_name="c", subcore_axis_name="s")

    @functools.partial(
        pl.kernel,
        mesh=mesh,
        out_type=jax.ShapeDtypeStruct((SEQ * B, D), jnp.float32),
        scratch_types=[
            pltpu.VMEM((NSLOT, CK), jnp.int32),
            pltpu.VMEM((NSLOT, CK, D), jnp.float32),
            pltpu.VMEM((BPW,), jnp.int32),
            pltpu.VMEM((BPW, D), jnp.float32),
        ] + [pltpu.SemaphoreType.DMA] * (1 + 3 * NSLOT),
    )
    def sc_gather(ids_t_hbm, pids_hbm, table_hbm, sparse_hbm, out_hbm,
                  idx_v, rows_v, pid_v, prow_v, psem, *sems):
        isem = sems[0:NSLOT]
        gsem = sems[NSLOT:2 * NSLOT]
        osem = sems[2 * NSLOT:3 * NSLOT]
        wid = lax.axis_index("s") * NC + lax.axis_index("c")
        base = wid * BPW
        u0 = wid * UNITS

        pltpu.sync_copy(pids_hbm.at[pl.ds(base, BPW)], pid_v)
        pltpu.async_copy(sparse_hbm.at[pid_v], prow_v, psem).wait()

        def scale_prow(i, carry):
            for j in range(D // L):
                prow_v[i, pl.ds(j * L, L)] = prow_v[i, pl.ds(j * L, L)] * SCALE
            return carry

        lax.fori_loop(0, BPW, scale_prow, 0)
        pltpu.sync_copy(prow_v, out_hbm.at[pl.ds(base, BPW)])

        def issue_idx(t, slot):
            u = u0 + t
            s = u // NCHUNK
            c = u % NCHUNK
            pltpu.async_copy(ids_t_hbm.at[s, pl.ds(c * CK, CK)],
                             idx_v.at[slot], isem[slot])

        def wait_idx(slot):
            pltpu.make_async_copy(ids_t_hbm.at[0, pl.ds(0, CK)],
                                  idx_v.at[slot], isem[slot]).wait()

        def issue_gather(slot):
            pltpu.async_copy(table_hbm.at[idx_v.at[slot]],
                             rows_v.at[slot], gsem[slot])

        def wait_gather(slot):
            pltpu.make_async_copy(table_hbm.at[pl.ds(0, CK)],
                                  rows_v.at[slot], gsem[slot]).wait()

        def scale_rows(slot):
            def srow(r, carry):
                for j in range(D // L):
                    rows_v[slot, r, pl.ds(j * L, L)] = (
                        rows_v[slot, r, pl.ds(j * L, L)] * SCALE)
                return carry
            lax.fori_loop(0, CK, srow, 0)

        def issue_out(t, slot):
            off = pl.multiple_of(B + (u0 + t) * CK, CK)
            pltpu.async_copy(rows_v.at[slot], out_hbm.at[pl.ds(off, CK)],
                             osem[slot])

        def wait_out(slot):
            pltpu.make_async_copy(rows_v.at[slot],
                                  out_hbm.at[pl.ds(0, CK)],
                                  osem[slot]).wait()

        for k in range(NSLOT - 1):
            issue_idx(k, k)
        for k in range(NSLOT - 2):
            wait_idx(k)
            issue_gather(k)

        def body(i4, carry):
            for p in range(NSLOT):
                t = i4 * NSLOT + p
                q = (p + NSLOT - 2) % NSLOT
                r = (p + NSLOT - 1) % NSLOT
                wait_gather(p)
                scale_rows(p)
                issue_out(t, p)

                @pl.when(t + NSLOT - 1 < UNITS)
                def _():
                    issue_idx(t + NSLOT - 1, r)

                @pl.when(t + NSLOT - 2 < UNITS)
                def _():
                    @pl.when(t >= 2)
                    def _():
                        wait_out(q)
                    wait_idx(q)
                    issue_gather(q)
            return carry

        lax.fori_loop(0, UNITS // NSLOT, body, 0)
        for t in range(NSLOT):
            wait_out((UNITS - NSLOT + t) % NSLOT)

    return sc_gather


_sc_gather = _make_sc_gather()


def kernel(input_ids, puzzle_identifiers, emb_table, sparse_table):
    ids_t = input_ids.T
    flat = _sc_gather(ids_t, puzzle_identifiers, emb_table, sparse_table)
    return flat.reshape(SEQ, B, D).swapaxes(0, 1)


# speedup vs baseline: 1.0050x; 1.0050x over previous
import functools
import math

import jax
import jax.numpy as jnp
from jax import lax
from jax.experimental import pallas as pl
from jax.experimental.pallas import tpu as pltpu
from jax.experimental.pallas import tpu_sc as plsc

VOCAB = 100000
D = 128
B = 4096
S = 200
SEQ = S + 1
SCALE = 1.0 / math.sqrt(D)

_info = plsc.get_sparse_core_info()
NC, NS, L = _info.num_cores, _info.num_subcores, _info.num_lanes
NW = NC * NS
BPW = B // NW
CK = 128
NCHUNK = B // CK
UNITS = S * NCHUNK // NW
NSLOT = 4


def _make_sc_gather():
    mesh = plsc.VectorSubcoreMesh(core_axis_name="c", subcore_axis_name="s")

    @functools.partial(
        pl.kernel,
        mesh=mesh,
        out_type=jax.ShapeDtypeStruct((SEQ * B, D), jnp.float32),
        scratch_types=[
            pltpu.VMEM((NSLOT, CK), jnp.int32),
            pltpu.VMEM((NSLOT, CK, D), jnp.float32),
            pltpu.VMEM((BPW,), jnp.int32),
            pltpu.VMEM((BPW, D), jnp.float32),
        ] + [pltpu.SemaphoreType.DMA] * (1 + 3 * NSLOT),
    )
    def sc_gather(ids_t_hbm, pids_hbm, table_hbm, sparse_hbm, out_hbm,
                  idx_v, rows_v, pid_v, prow_v, psem, *sems):
        isem = sems[0:NSLOT]
        gsem = sems[NSLOT:2 * NSLOT]
        osem = sems[2 * NSLOT:3 * NSLOT]
        wid = lax.axis_index("s") * NC + lax.axis_index("c")
        base = wid * BPW
        u0 = wid * UNITS

        pltpu.sync_copy(pids_hbm.at[pl.ds(base, BPW)], pid_v)
        pltpu.async_copy(sparse_hbm.at[pid_v], prow_v, psem).wait()

        def scale_prow(i, carry):
            for j in range(D // L):
                prow_v[i, pl.ds(j * L, L)] = prow_v[i, pl.ds(j * L, L)] * SCALE
            return carry

        lax.fori_loop(0, BPW, scale_prow, 0)
        pltpu.sync_copy(prow_v, out_hbm.at[pl.ds(base, BPW)])

        def issue_idx(t, slot):
            u = u0 + t
            s = u // NCHUNK
            c = u % NCHUNK
            pltpu.async_copy(ids_t_hbm.at[s, pl.ds(c * CK, CK)],
                             idx_v.at[slot], isem[slot])

        def wait_idx(slot):
            pltpu.make_async_copy(ids_t_hbm.at[0, pl.ds(0, CK)],
                                  idx_v.at[slot], isem[slot]).wait()

        def issue_gather(slot):
            pltpu.async_copy(table_hbm.at[idx_v.at[slot]],
                             rows_v.at[slot], gsem[slot])

        def wait_gather(slot):
            pltpu.make_async_copy(table_hbm.at[pl.ds(0, CK)],
                                  rows_v.at[slot], gsem[slot]).wait()

        def scale_rows(slot):
            def srow(r, carry):
                for j in range(D // L):
                    rows_v[slot, r, pl.ds(j * L, L)] = (
                        rows_v[slot, r, pl.ds(j * L, L)] * SCALE)
                return carry
            lax.fori_loop(0, CK, srow, 0)

        def issue_out(t, slot):
            off = pl.multiple_of(B + (u0 + t) * CK, CK)
            pltpu.async_copy(rows_v.at[slot], out_hbm.at[pl.ds(off, CK)],
                             osem[slot])

        def wait_out(slot):
            pltpu.make_async_copy(rows_v.at[slot],
                                  out_hbm.at[pl.ds(0, CK)],
                                  osem[slot]).wait()

        for k in range(NSLOT - 1):
            issue_idx(k, k)
        for k in range(NSLOT - 2):
            wait_idx(k)
            issue_gather(k)

        def body(i4, carry):
            for p in range(NSLOT):
                t = i4 * NSLOT + p
                q = (p + NSLOT - 2) % NSLOT
                r = (p + NSLOT - 1) % NSLOT
                wait_gather(p)
                scale_rows(p)
                issue_out(t, p)

                @pl.when(t + NSLOT - 1 < UNITS)
                def _():
                    issue_idx(t + NSLOT - 1, r)

                @pl.when(t + NSLOT - 2 < UNITS)
                def _():
                    @pl.when(t >= 2)
                    def _():
                        wait_out(q)
                    wait_idx(q)
                    issue_gather(q)
            return carry

        lax.fori_loop(0, UNITS // NSLOT, body, 0)
        for t in range(NSLOT):
            wait_out((UNITS - NSLOT + t) % NSLOT)

    return sc_gather


_sc_gather = _make_sc_gather()


def kernel(input_ids, puzzle_identifiers, emb_table, sparse_table):
    ids_t = input_ids.T
    flat = _sc_gather(ids_t, puzzle_identifiers, emb_table, sparse_table)
    return flat.reshape(SEQ, B, D).swapaxes(0, 1)


# speedup vs baseline: 1.0057x; 1.0007x over previous
import functools
import math

import jax
import jax.numpy as jnp
from jax import lax
from jax.experimental import pallas as pl
from jax.experimental.pallas import tpu as pltpu
from jax.experimental.pallas import tpu_sc as plsc

VOCAB = 100000
D = 128
B = 4096
S = 200
SEQ = S + 1
SCALE = 1.0 / math.sqrt(D)

_info = plsc.get_sparse_core_info()
NC, NS, L = _info.num_cores, _info.num_subcores, _info.num_lanes
NW = NC * NS
BPW = B // NW
CK = 128
NCHUNK = B // CK
UNITS = S * NCHUNK // NW
NSLOT = 4


def _make_sc_gather():
    mesh = plsc.VectorSubcoreMesh(core_axis_name="c", subcore_axis_name="s")

    @functools.partial(
        pl.kernel,
        mesh=mesh,
        out_type=jax.ShapeDtypeStruct((SEQ * B, D), jnp.float32),
        scratch_types=[
            pltpu.VMEM((NSLOT, CK), jnp.int32),
            pltpu.VMEM((NSLOT, CK, D), jnp.float32),
            pltpu.VMEM((BPW,), jnp.int32),
            pltpu.VMEM((BPW, D), jnp.float32),
        ] + [pltpu.SemaphoreType.DMA] * (1 + 3 * NSLOT),
    )
    def sc_gather(ids_t_hbm, pids_hbm, table_hbm, sparse_hbm, out_hbm,
                  idx_v, rows_v, pid_v, prow_v, psem, *sems):
        isem = sems[0:NSLOT]
        gsem = sems[NSLOT:2 * NSLOT]
        osem = sems[2 * NSLOT:3 * NSLOT]
        wid = lax.axis_index("s") * NC + lax.axis_index("c")
        base = wid * BPW
        u0 = wid * UNITS

        pltpu.sync_copy(pids_hbm.at[pl.ds(base, BPW)], pid_v)
        pltpu.async_copy(sparse_hbm.at[pid_v], prow_v, psem).wait()

        def scale_prow(i, carry):
            for j in range(D // L):
                prow_v[i, pl.ds(j * L, L)] = prow_v[i, pl.ds(j * L, L)] * SCALE
            return carry

        lax.fori_loop(0, BPW, scale_prow, 0)
        pltpu.sync_copy(prow_v, out_hbm.at[pl.ds(base, BPW)])

        def issue_idx(t, slot):
            u = u0 + t
            s = u // NCHUNK
            c = u % NCHUNK
            pltpu.async_copy(ids_t_hbm.at[s, pl.ds(c * CK, CK)],
                             idx_v.at[slot], isem[slot])

        def wait_idx(slot):
            pltpu.make_async_copy(ids_t_hbm.at[0, pl.ds(0, CK)],
                                  idx_v.at[slot], isem[slot]).wait()

        def issue_gather(slot):
            pltpu.async_copy(table_hbm.at[idx_v.at[slot]],
                             rows_v.at[slot], gsem[slot])

        def wait_gather(slot):
            pltpu.make_async_copy(table_hbm.at[idx_v.at[slot]],
                                  rows_v.at[slot], gsem[slot]).wait()

        def scale_rows(slot):
            def srow(r, carry):
                for j in range(D // L):
                    rows_v[slot, r, pl.ds(j * L, L)] = (
                        rows_v[slot, r, pl.ds(j * L, L)] * SCALE)
                return carry
            lax.fori_loop(0, CK, srow, 0)

        def issue_out(t, slot):
            off = pl.multiple_of(B + (u0 + t) * CK, CK)
            pltpu.async_copy(rows_v.at[slot], out_hbm.at[pl.ds(off, CK)],
                             osem[slot])

        def wait_out(slot):
            pltpu.make_async_copy(rows_v.at[slot],
                                  out_hbm.at[pl.ds(0, CK)],
                                  osem[slot]).wait()

        for k in range(NSLOT - 1):
            issue_idx(k, k)
        for k in range(NSLOT - 2):
            wait_idx(k)
            issue_gather(k)

        def body(i4, carry):
            for p in range(NSLOT):
                t = i4 * NSLOT + p
                q = (p + NSLOT - 2) % NSLOT
                r = (p + NSLOT - 1) % NSLOT
                wait_gather(p)
                scale_rows(p)
                issue_out(t, p)

                @pl.when(t + NSLOT - 1 < UNITS)
                def _():
                    issue_idx(t + NSLOT - 1, r)

                @pl.when(t + NSLOT - 2 < UNITS)
                def _():
                    @pl.when(t >= 2)
                    def _():
                        wait_out(q)
                    wait_idx(q)
                    issue_gather(q)
            return carry

        lax.fori_loop(0, UNITS // NSLOT, body, 0)
        for t in range(NSLOT):
            wait_out((UNITS - NSLOT + t) % NSLOT)

    return sc_gather


_sc_gather = _make_sc_gather()


def kernel(input_ids, puzzle_identifiers, emb_table, sparse_table):
    ids_t = input_ids.T
    flat = _sc_gather(ids_t, puzzle_identifiers, emb_table, sparse_table)
    return flat.reshape(SEQ, B, D).swapaxes(0, 1)
